# baseline (speedup 1.0000x reference)
import functools

import jax
import jax.numpy as jnp
from jax import lax
from jax.experimental import pallas as pl
from jax.experimental.pallas import tpu as pltpu
from jax.experimental.pallas import tpu_sc as plsc

VOCAB_ = 100000
EMB_ = 128
CTX2_ = 50
HID_ = 128
VTILE_ = 20480
NT_ = (VOCAB_ + VTILE_ - 1) // VTILE_
_PREC = jax.lax.Precision.DEFAULT

_SC_MESH = plsc.VectorSubcoreMesh(core_axis_name="c", subcore_axis_name="s")


@functools.partial(
    pl.kernel,
    out_type=jax.ShapeDtypeStruct((CTX2_, EMB_), jnp.float32),
    mesh=_SC_MESH,
    scratch_types=[
        pltpu.VMEM((CTX2_,), jnp.int32),
        pltpu.VMEM((CTX2_, EMB_), jnp.float32),
        pltpu.SemaphoreType.DMA,
    ],
)
def _sc_gather(idx_hbm, table_hbm, out_hbm, idx_v, rows_v, sem):
    @pl.when((lax.axis_index("c") == 0) & (lax.axis_index("s") == 0))
    def _():
        pltpu.sync_copy(idx_hbm, idx_v)
        pltpu.async_copy(table_hbm.at[idx_v], rows_v, sem).wait()
        pltpu.sync_copy(rows_v, out_hbm)


def _fused_kernel(flat_ref, w1_ref, b1_ref, w2_ref, b2_ref,
                  logits_ref, lse_ref, h_ref, m_ref, s_ref):
    t = pl.program_id(0)

    @pl.when(t == 0)
    def _fc1():
        p = jax.lax.dot_general(flat_ref[...], w1_ref[...],
                                (((1,), (1,)), ((), ())),
                                precision=_PREC,
                                preferred_element_type=jnp.float32)
        h_ref[...] = jnp.maximum(p + b1_ref[...], 0.0)

    h = h_ref[...]
    l = jax.lax.dot_general(h, w2_ref[...], (((1,), (1,)), ((), ())),
                            precision=_PREC,
                            preferred_element_type=jnp.float32)
    l = l + b2_ref[...]
    logits_ref[...] = l

    col = t * VTILE_ + jax.lax.broadcasted_iota(jnp.int32, (1, VTILE_), 1)
    lm = jnp.where(col < VOCAB_, l, -jnp.inf)
    tmax = jnp.max(lm)

    @pl.when(t == 0)
    def _init():
        m_ref[0, 0] = tmax
        s_ref[0, 0] = jnp.sum(jnp.exp(lm - tmax))

    @pl.when(t > 0)
    def _acc():
        m_old = m_ref[0, 0]
        m_new = jnp.maximum(m_old, tmax)
        s_ref[0, 0] = (s_ref[0, 0] * jnp.exp(m_old - m_new)
                       + jnp.sum(jnp.exp(lm - m_new)))
        m_ref[0, 0] = m_new

    @pl.when(t == pl.num_programs(0) - 1)
    def _fin():
        lse_ref[...] = jnp.full((1, 1), m_ref[0, 0] + jnp.log(s_ref[0, 0]),
                                dtype=jnp.float32)


def _norm_kernel(logits_ref, lse_ref, out_ref):
    out_ref[...] = logits_ref[...] - lse_ref[...]


@functools.partial(jax.jit, static_argnames=("interpret",))
def _run(x, emb, W1, b1, W2, b2, interpret=False):
    b1r = b1.reshape(1, HID_)
    b2r = b2.reshape(1, VOCAB_)

    rows = _sc_gather(x, emb)
    flat = rows.reshape(1, CTX2_ * EMB_)

    logits, lse = pl.pallas_call(
        _fused_kernel,
        grid=(NT_,),
        in_specs=[
            pl.BlockSpec((1, CTX2_ * EMB_), lambda t: (0, 0)),
            pl.BlockSpec((HID_, CTX2_ * EMB_), lambda t: (0, 0)),
            pl.BlockSpec((1, HID_), lambda t: (0, 0)),
            pl.BlockSpec((VTILE_, HID_), lambda t: (t, 0)),
            pl.BlockSpec((1, VTILE_), lambda t: (0, t)),
        ],
        out_specs=[
            pl.BlockSpec((1, VTILE_), lambda t: (0, t)),
            pl.BlockSpec((1, 1), lambda t: (0, 0)),
        ],
        scratch_shapes=[
            pltpu.VMEM((1, HID_), jnp.float32),
            pltpu.SMEM((1, 1), jnp.float32),
            pltpu.SMEM((1, 1), jnp.float32),
        ],
        out_shape=[
            jax.ShapeDtypeStruct((1, VOCAB_), jnp.float32),
            jax.ShapeDtypeStruct((1, 1), jnp.float32),
        ],
        interpret=interpret,
    )(flat, W1, b1r, W2, b2r)

    out = pl.pallas_call(
        _norm_kernel,
        interpret=interpret,
        input_output_aliases={0: 0},
        out_shape=jax.ShapeDtypeStruct((1, VOCAB_), jnp.float32),
    )(logits, lse)

    return out


def kernel(x, emb, W1, b1, W2, b2):
    return _run(x, emb, W1, b1, W2, b2)


# speedup vs baseline: 1.6569x; 1.6569x over previous
import functools

import jax
import jax.numpy as jnp
from jax.experimental import pallas as pl
from jax.experimental.pallas import tpu as pltpu

VOCAB_ = 100000
EMB_ = 128
CTX2_ = 50
HID_ = 128
VTILE_ = 20480
NT_ = (VOCAB_ + VTILE_ - 1) // VTILE_
_PREC = jax.lax.Precision.DEFAULT


def _fused_kernel(x_ref, emb_hbm, w1_ref, b1_ref, w2_ref, b2_ref,
                  out_ref, flat_ref, h_ref, lbuf_ref, m_ref, s_ref, sem):
    i = pl.program_id(0)

    @pl.when(i == 0)
    def _gather_fc1():
        copies = []
        for j in range(CTX2_):
            c = pltpu.make_async_copy(
                emb_hbm.at[pl.ds(x_ref[j], 1), :],
                flat_ref.at[:, pl.ds(j * EMB_, EMB_)],
                sem)
            c.start()
            copies.append(c)
        for c in copies:
            c.wait()
        p = jax.lax.dot_general(flat_ref[...], w1_ref[...],
                                (((1,), (1,)), ((), ())),
                                precision=_PREC,
                                preferred_element_type=jnp.float32)
        h_ref[...] = jnp.maximum(p + b1_ref[...], 0.0)

    @pl.when(i < NT_)
    def _phase1():
        t = i
        h = h_ref[...]
        l = jax.lax.dot_general(h, w2_ref[...], (((1,), (1,)), ((), ())),
                                precision=_PREC,
                                preferred_element_type=jnp.float32)
        l = l + b2_ref[...]
        lbuf_ref[:, pl.ds(t * VTILE_, VTILE_)] = l
        out_ref[...] = l

        col = t * VTILE_ + jax.lax.broadcasted_iota(jnp.int32, (1, VTILE_), 1)
        lm = jnp.where(col < VOCAB_, l, -jnp.inf)
        tmax = jnp.max(lm)

        @pl.when(t == 0)
        def _init():
            m_ref[0, 0] = tmax
            s_ref[0, 0] = jnp.sum(jnp.exp(lm - tmax))

        @pl.when(t > 0)
        def _acc():
            m_old = m_ref[0, 0]
            m_new = jnp.maximum(m_old, tmax)
            s_ref[0, 0] = (s_ref[0, 0] * jnp.exp(m_old - m_new)
                           + jnp.sum(jnp.exp(lm - m_new)))
            m_ref[0, 0] = m_new

    @pl.when(i >= NT_)
    def _phase2():
        t = i - NT_
        lse = m_ref[0, 0] + jnp.log(s_ref[0, 0])
        out_ref[...] = lbuf_ref[:, pl.ds(t * VTILE_, VTILE_)] - lse


@functools.partial(jax.jit, static_argnames=("interpret",))
def _run(x, emb, W1, b1, W2, b2, interpret=False):
    b1r = b1.reshape(1, HID_)
    b2r = b2.reshape(1, VOCAB_)

    out = pl.pallas_call(
        _fused_kernel,
        grid_spec=pltpu.PrefetchScalarGridSpec(
            num_scalar_prefetch=1,
            grid=(2 * NT_,),
            in_specs=[
                pl.BlockSpec(memory_space=pltpu.MemorySpace.HBM),
                pl.BlockSpec((HID_, CTX2_ * EMB_), lambda i, xr: (0, 0)),
                pl.BlockSpec((1, HID_), lambda i, xr: (0, 0)),
                pl.BlockSpec((VTILE_, HID_),
                             lambda i, xr: (jnp.minimum(i, NT_ - 1), 0)),
                pl.BlockSpec((1, VTILE_),
                             lambda i, xr: (0, jnp.minimum(i, NT_ - 1))),
            ],
            out_specs=pl.BlockSpec((1, VTILE_), lambda i, xr: (0, i % NT_)),
            scratch_shapes=[
                pltpu.VMEM((1, CTX2_ * EMB_), jnp.float32),
                pltpu.VMEM((1, HID_), jnp.float32),
                pltpu.VMEM((1, NT_ * VTILE_), jnp.float32),
                pltpu.SMEM((1, 1), jnp.float32),
                pltpu.SMEM((1, 1), jnp.float32),
                pltpu.SemaphoreType.DMA,
            ],
        ),
        out_shape=jax.ShapeDtypeStruct((1, VOCAB_), jnp.float32),
        interpret=interpret,
    )(x, emb, W1, b1r, W2, b2r)

    return out


def kernel(x, emb, W1, b1, W2, b2):
    return _run(x, emb, W1, b1, W2, b2)


# speedup vs baseline: 1.6750x; 1.0109x over previous
import functools

import jax
import jax.numpy as jnp
from jax.experimental import pallas as pl
from jax.experimental.pallas import tpu as pltpu

VOCAB_ = 100000
EMB_ = 128
CTX2_ = 50
HID_ = 128
VTILE_ = 20480
NT_ = (VOCAB_ + VTILE_ - 1) // VTILE_
_PREC = jax.lax.Precision.DEFAULT


def _fused_kernel(x_ref, emb_hbm, w1_ref, b1_ref, w2_ref, b2_ref,
                  out_ref, flat_ref, h_ref, lbuf_ref, m_ref, s_ref, sem):
    i = pl.program_id(0)

    @pl.when(i == 0)
    def _gather_fc1():
        copies = []
        for j in range(CTX2_):
            c = pltpu.make_async_copy(
                emb_hbm.at[pl.ds(x_ref[j], 1), :],
                flat_ref.at[:, pl.ds(j * EMB_, EMB_)],
                sem)
            c.start()
            copies.append(c)
        for c in copies:
            c.wait()
        p = jax.lax.dot_general(flat_ref[...], w1_ref[...],
                                (((1,), (1,)), ((), ())),
                                precision=_PREC,
                                preferred_element_type=jnp.float32)
        h_ref[...] = jnp.maximum(p + b1_ref[...], 0.0)

    @pl.when(i < NT_)
    def _phase1():
        t = i
        h = h_ref[...]
        l = jax.lax.dot_general(h, w2_ref[...], (((1,), (1,)), ((), ())),
                                precision=_PREC,
                                preferred_element_type=jnp.float32)
        l = l + b2_ref[...]
        lbuf_ref[:, pl.ds(t * VTILE_, VTILE_)] = l

        col = t * VTILE_ + jax.lax.broadcasted_iota(jnp.int32, (1, VTILE_), 1)
        lm = jnp.where(col < VOCAB_, l, -jnp.inf)
        tmax = jnp.max(lm)

        @pl.when(t == 0)
        def _init():
            m_ref[0, 0] = tmax
            s_ref[0, 0] = jnp.sum(jnp.exp(lm - tmax))

        @pl.when(t > 0)
        def _acc():
            m_old = m_ref[0, 0]
            m_new = jnp.maximum(m_old, tmax)
            s_ref[0, 0] = (s_ref[0, 0] * jnp.exp(m_old - m_new)
                           + jnp.sum(jnp.exp(lm - m_new)))
            m_ref[0, 0] = m_new

    @pl.when(i >= NT_)
    def _phase2():
        t = i - NT_
        lse = m_ref[0, 0] + jnp.log(s_ref[0, 0])
        out_ref[...] = lbuf_ref[:, pl.ds(t * VTILE_, VTILE_)] - lse


@functools.partial(jax.jit, static_argnames=("interpret",))
def _run(x, emb, W1, b1, W2, b2, interpret=False):
    b1r = b1.reshape(1, HID_)
    b2r = b2.reshape(1, VOCAB_)

    out = pl.pallas_call(
        _fused_kernel,
        grid_spec=pltpu.PrefetchScalarGridSpec(
            num_scalar_prefetch=1,
            grid=(2 * NT_,),
            in_specs=[
                pl.BlockSpec(memory_space=pltpu.MemorySpace.HBM),
                pl.BlockSpec((HID_, CTX2_ * EMB_), lambda i, xr: (0, 0)),
                pl.BlockSpec((1, HID_), lambda i, xr: (0, 0)),
                pl.BlockSpec((VTILE_, HID_),
                             lambda i, xr: (jnp.minimum(i, NT_ - 1), 0)),
                pl.BlockSpec((1, VTILE_),
                             lambda i, xr: (0, jnp.minimum(i, NT_ - 1))),
            ],
            out_specs=pl.BlockSpec(
                (1, VTILE_),
                lambda i, xr: (0, jnp.where(i < NT_, 0, i - NT_))),
            scratch_shapes=[
                pltpu.VMEM((1, CTX2_ * EMB_), jnp.float32),
                pltpu.VMEM((1, HID_), jnp.float32),
                pltpu.VMEM((1, NT_ * VTILE_), jnp.float32),
                pltpu.SMEM((1, 1), jnp.float32),
                pltpu.SMEM((1, 1), jnp.float32),
                pltpu.SemaphoreType.DMA,
            ],
        ),
        out_shape=jax.ShapeDtypeStruct((1, VOCAB_), jnp.float32),
        interpret=interpret,
    )(x, emb, W1, b1r, W2, b2r)

    return out


def kernel(x, emb, W1, b1, W2, b2):
    return _run(x, emb, W1, b1, W2, b2)
